# baseline (speedup 1.0000x reference)
import functools

import jax
import jax.numpy as jnp
from jax import lax
from jax.experimental import pallas as pl
from jax.experimental.pallas import tpu as pltpu
from jax.experimental.pallas import tpu_sc as plsc

BATCH = 16384
EMBED_K = 64
LANES = 16

F_SC = 8192
F_TC = BATCH - F_SC

_info = plsc.get_sparse_core_info()
NC, NS = _info.num_cores, _info.num_subcores
NW = NC * NS
B_PER_W = F_SC // NW
CHUNK = 128
NCHUNK = B_PER_W // CHUNK

_mesh = plsc.VectorSubcoreMesh(core_axis_name="c", subcore_axis_name="s")


@functools.partial(
    pl.kernel,
    out_type=(
        jax.ShapeDtypeStruct((F_SC,), jnp.float32),
        jax.ShapeDtypeStruct((F_SC, EMBED_K), jnp.float32),
        jax.ShapeDtypeStruct((F_SC, EMBED_K), jnp.float32),
    ),
    mesh=_mesh,
    compiler_params=pltpu.CompilerParams(
        needs_layout_passes=False, use_tc_tiling_on_sc=False),
    scratch_types=[
        pltpu.VMEM((NCHUNK, CHUNK), jnp.int32),
        pltpu.VMEM((NCHUNK, CHUNK), jnp.int32),
        pltpu.VMEM((B_PER_W, EMBED_K), jnp.float32),
        pltpu.VMEM((B_PER_W, EMBED_K), jnp.float32),
        pltpu.VMEM((B_PER_W,), jnp.float32),
        pltpu.SemaphoreType.DMA,
        pltpu.SemaphoreType.DMA,
    ],
)
def _sc_kernel(users_hbm, items_hbm, gu_hbm, gi_hbm,
               xui_hbm, gu_out_hbm, gi_out_hbm,
               idx_u, idx_i, rows_u, rows_i, xui_v, sem_u, sem_i):
    wid = lax.axis_index("s") * NC + lax.axis_index("c")
    base = wid * B_PER_W

    pltpu.sync_copy(users_hbm.at[wid], idx_u)
    pltpu.sync_copy(items_hbm.at[wid], idx_i)

    for j in range(NCHUNK):
        pltpu.async_copy(gu_hbm.at[idx_u.at[j]],
                         rows_u.at[pl.ds(j * CHUNK, CHUNK)], sem_u)
    for j in range(NCHUNK):
        pltpu.async_copy(gi_hbm.at[idx_i.at[j]],
                         rows_i.at[pl.ds(j * CHUNK, CHUNK)], sem_i)
    for j in range(NCHUNK):
        pltpu.make_async_copy(gu_hbm.at[idx_u.at[j]],
                              rows_u.at[pl.ds(j * CHUNK, CHUNK)], sem_u).wait()
    pltpu.sync_copy(rows_u, gu_out_hbm.at[pl.ds(base, B_PER_W)])
    for j in range(NCHUNK):
        pltpu.make_async_copy(gi_hbm.at[idx_i.at[j]],
                              rows_i.at[pl.ds(j * CHUNK, CHUNK)], sem_i).wait()
    pltpu.sync_copy(rows_i, gi_out_hbm.at[pl.ds(base, B_PER_W)])

    lane_iota = jnp.arange(LANES, dtype=jnp.int32)

    def group_body(g, _):
        rbase = g * LANES
        acc = jnp.zeros((LANES,), jnp.float32)
        for rr in range(LANES):
            r = rbase + rr
            s = jnp.zeros((LANES,), jnp.float32)
            for c in range(EMBED_K // LANES):
                u = rows_u[r, pl.ds(c * LANES, LANES)]
                v = rows_i[r, pl.ds(c * LANES, LANES)]
                s = s + u * v
            acc = jnp.where(lane_iota == rr, jnp.sum(s), acc)
        xui_v[pl.ds(rbase, LANES)] = acc
        return 0

    lax.fori_loop(0, B_PER_W // LANES, group_body, 0)
    pltpu.sync_copy(xui_v, xui_hbm.at[pl.ds(base, B_PER_W)])


CH = 512
NSTEP = F_TC // CH


def _tc_body(users_smem, items_smem, gu_any, gi_any,
             xui_ref, gu_out, gi_out, rows_u, rows_i, sem_u, sem_i):
    step = pl.program_id(0)
    cbase = step * CH

    def issue(j, _):
        r_u = users_smem[cbase + j]
        pltpu.make_async_copy(gu_any.at[r_u], rows_u.at[j], sem_u).start()
        r_i = items_smem[cbase + j]
        pltpu.make_async_copy(gi_any.at[r_i], rows_i.at[j], sem_i).start()
        return 0

    lax.fori_loop(0, CH, issue, 0, unroll=8)
    pltpu.make_async_copy(gu_any.at[pl.ds(0, CH)], rows_u, sem_u).wait()
    pltpu.make_async_copy(gi_any.at[pl.ds(0, CH)], rows_i, sem_i).wait()

    u = rows_u[...]
    v = rows_i[...]
    gu_out[...] = u
    gi_out[...] = v
    xui_ref[...] = jnp.sum(u * v, axis=1)


_tc_call = pl.pallas_call(
    _tc_body,
    grid_spec=pltpu.PrefetchScalarGridSpec(
        num_scalar_prefetch=2,
        grid=(NSTEP,),
        in_specs=[
            pl.BlockSpec(memory_space=pl.ANY),
            pl.BlockSpec(memory_space=pl.ANY),
        ],
        out_specs=[
            pl.BlockSpec((CH,), lambda i, users, items: (i,)),
            pl.BlockSpec((CH, EMBED_K), lambda i, users, items: (i, 0)),
            pl.BlockSpec((CH, EMBED_K), lambda i, users, items: (i, 0)),
        ],
        scratch_shapes=[
            pltpu.VMEM((CH, EMBED_K), jnp.float32),
            pltpu.VMEM((CH, EMBED_K), jnp.float32),
            pltpu.SemaphoreType.DMA,
            pltpu.SemaphoreType.DMA,
        ],
    ),
    out_shape=[
        jax.ShapeDtypeStruct((F_TC,), jnp.float32),
        jax.ShapeDtypeStruct((F_TC, EMBED_K), jnp.float32),
        jax.ShapeDtypeStruct((F_TC, EMBED_K), jnp.float32),
    ],
)


def kernel(users, items, Gu, Gi):
    users_sc = users[:F_SC].reshape(NW, NCHUNK, CHUNK)
    items_sc = items[:F_SC].reshape(NW, NCHUNK, CHUNK)
    xui_b, gu_b, gi_b = _tc_call(users[F_SC:], items[F_SC:], Gu, Gi)
    xui_a, gu_a, gi_a = _sc_kernel(users_sc, items_sc, Gu, Gi)
    xui = jnp.concatenate([xui_a, xui_b], axis=0)
    gamma_u = jnp.concatenate([gu_a, gu_b], axis=0)
    gamma_i = jnp.concatenate([gi_a, gi_b], axis=0)
    return (xui, gamma_u, gamma_i)


# speedup vs baseline: 2.1786x; 2.1786x over previous
import functools

import jax
import jax.numpy as jnp
from jax import lax
from jax.experimental import pallas as pl
from jax.experimental.pallas import tpu as pltpu
from jax.experimental.pallas import tpu_sc as plsc

BATCH = 16384
EMBED_K = 64
LANES = 16

_info = plsc.get_sparse_core_info()
NC, NS = _info.num_cores, _info.num_subcores
NW = NC * NS
B_PER_W = BATCH // NW
NPASS = 2
P_ROWS = B_PER_W // NPASS
WINDOW = 64

_mesh = plsc.VectorSubcoreMesh(core_axis_name="c", subcore_axis_name="s")


@functools.partial(
    pl.kernel,
    out_type=(
        jax.ShapeDtypeStruct((BATCH,), jnp.float32),
        jax.ShapeDtypeStruct((BATCH, EMBED_K), jnp.float32),
        jax.ShapeDtypeStruct((BATCH, EMBED_K), jnp.float32),
    ),
    mesh=_mesh,
    compiler_params=pltpu.CompilerParams(needs_layout_passes=False),
    scratch_types=[
        pltpu.VMEM((B_PER_W,), jnp.int32),
        pltpu.VMEM((B_PER_W,), jnp.int32),
        pltpu.VMEM((P_ROWS, EMBED_K), jnp.float32),
        pltpu.VMEM((P_ROWS, EMBED_K), jnp.float32),
        pltpu.VMEM((B_PER_W,), jnp.float32),
        pltpu.SemaphoreType.DMA,
        pltpu.SemaphoreType.DMA,
    ],
)
def _bpr_kernel(users_hbm, items_hbm, gu_hbm, gi_hbm,
                xui_hbm, gu_out_hbm, gi_out_hbm,
                idx_u, idx_i, rows_u, rows_i, xui_v, sem_u, sem_i):
    wid = lax.axis_index("s") * NC + lax.axis_index("c")
    base = wid * B_PER_W

    pltpu.sync_copy(users_hbm.at[pl.ds(base, B_PER_W)], idx_u)
    pltpu.sync_copy(items_hbm.at[pl.ds(base, B_PER_W)], idx_i)

    def drain_one(sem):
        pltpu.make_async_copy(gu_hbm.at[0], rows_u.at[0], sem).wait()

    lane_iota = jnp.arange(LANES, dtype=jnp.int32)
    gwin = WINDOW // LANES

    for p in range(NPASS):
        pbase = p * P_ROWS

        def fetch_group(g, _):
            gb = g * LANES
            vu = idx_u[pl.ds(pbase + gb, LANES)]
            vi = idx_i[pl.ds(pbase + gb, LANES)]
            for rr in range(LANES):
                pltpu.async_copy(gu_hbm.at[vu[rr]], rows_u.at[gb + rr], sem_u)
                pltpu.async_copy(gi_hbm.at[vi[rr]], rows_i.at[gb + rr], sem_i)

            @pl.when(g >= gwin)
            def _():
                for _ in range(LANES):
                    drain_one(sem_u)
                    drain_one(sem_i)

            return 0

        lax.fori_loop(0, P_ROWS // LANES, fetch_group, 0)
        for _ in range(WINDOW):
            drain_one(sem_u)
            drain_one(sem_i)

        pltpu.sync_copy(rows_u, gu_out_hbm.at[pl.ds(base + pbase, P_ROWS)])
        pltpu.sync_copy(rows_i, gi_out_hbm.at[pl.ds(base + pbase, P_ROWS)])

        def group_body(g, _):
            rbase = g * LANES
            acc = jnp.zeros((LANES,), jnp.float32)
            for rr in range(LANES):
                r = rbase + rr
                s = jnp.zeros((LANES,), jnp.float32)
                for c in range(EMBED_K // LANES):
                    u = rows_u[r, pl.ds(c * LANES, LANES)]
                    v = rows_i[r, pl.ds(c * LANES, LANES)]
                    s = s + u * v
                acc = jnp.where(lane_iota == rr, jnp.sum(s), acc)
            xui_v[pl.ds(pbase + rbase, LANES)] = acc
            return 0

        lax.fori_loop(0, P_ROWS // LANES, group_body, 0)

    pltpu.sync_copy(xui_v, xui_hbm.at[pl.ds(base, B_PER_W)])


def kernel(users, items, Gu, Gi):
    return _bpr_kernel(users, items, Gu, Gi)
